# Initial kernel scaffold; baseline (speedup 1.0000x reference)
import jax
import jax.numpy as jnp
from jax.experimental import pallas as pl


def kernel(x_single, x_multi, x_multi_vals, x_multi_lens, single_tables, multi_tables):
    raise NotImplementedError("write your pallas kernel here")


# speedup vs baseline: 3.4124x; 3.4124x over previous
import functools

import jax
import jax.numpy as jnp
from jax import lax
from jax.experimental import pallas as pl
from jax.experimental.pallas import tpu as pltpu
from jax.experimental.pallas import tpu_sc as plsc

NC = 2
NS = 16
LANES = 16
NW = NC * NS

IDXW = 80


def _make_kernel(B, NSF, NMF, L, V, D):
    RB = B // NW
    GB = 16
    NG = RB // GB
    KPG = GB * L // IDXW

    mesh = plsc.VectorSubcoreMesh(core_axis_name="c", subcore_axis_name="s")

    @functools.partial(
        pl.kernel,
        out_type=(
            jax.ShapeDtypeStruct((NSF, B, D), jnp.float32),
            jax.ShapeDtypeStruct((NMF, B // GB, D * GB), jnp.float32),
        ),
        mesh=mesh,
        compiler_params=pltpu.CompilerParams(
            needs_layout_passes=False, use_tc_tiling_on_sc=False),
        scratch_types=[
            pltpu.VMEM((RB,), jnp.int32),
            pltpu.VMEM((RB, D), jnp.float32),
            pltpu.VMEM((GB * L,), jnp.int32),
            pltpu.VMEM((GB * L,), jnp.float32),
            pltpu.VMEM((GB,), jnp.float32),
            pltpu.VMEM((GB * L, D), jnp.float32),
            pltpu.VMEM((D * GB,), jnp.float32),
            pltpu.SemaphoreType.DMA,
        ],
    )
    def k(ts_hbm, tm_hbm, idxs_hbm, idxm_hbm, vals_hbm, len_hbm,
          outs_hbm, outm_hbm,
          sidx_v, srows_v, midx_v, vals_v, len_v, stage_v, pt_v, sem):
        wid = lax.axis_index("s") * NC + lax.axis_index("c")
        base = pl.multiple_of(wid * RB, RB)

        iota = lax.iota(jnp.int32, LANES)
        rowb = iota * L

        for f in range(NSF):
            pltpu.sync_copy(idxs_hbm.at[pl.ds(f * B + base, RB)], sidx_v)
            pltpu.async_copy(ts_hbm.at[sidx_v], srows_v, sem).wait()
            pltpu.sync_copy(srows_v, outs_hbm.at[f, pl.ds(base, RB)])

        for f in range(NMF):
            pad = f * V

            def group_body(g, carry, f=f, pad=pad):
                gbase = pl.multiple_of(base + g * GB, GB)
                goff = pl.multiple_of(gbase * L + f * B * L, GB * L)
                pltpu.sync_copy(idxm_hbm.at[pl.ds(goff, GB * L)], midx_v)
                pltpu.sync_copy(vals_hbm.at[pl.ds(goff, GB * L)], vals_v)
                pltpu.sync_copy(
                    len_hbm.at[pl.ds(pl.multiple_of(f * B + gbase, GB), GB)],
                    len_v)
                copies = []
                for kk in range(KPG):
                    copies.append(pltpu.async_copy(
                        tm_hbm.at[midx_v.at[pl.ds(kk * IDXW, IDXW)]],
                        stage_v.at[pl.ds(kk * IDXW, IDXW)], sem))
                for c in copies:
                    c.wait()

                def l_body(l, acc, pad=pad):
                    jv = rowb + l
                    iv = plsc.load_gather(midx_v, [jv])
                    wv = plsc.load_gather(vals_v, [jv])
                    wv = jnp.where(iv != pad, wv, 0.0)
                    out = []
                    for d in range(D):
                        dcol = jnp.full((LANES,), d, jnp.int32)
                        gv = plsc.load_gather(stage_v, [jv, dcol])
                        out.append(acc[d] + wv * gv)
                    return tuple(out)

                acc = lax.fori_loop(
                    0, L, l_body,
                    tuple(jnp.zeros((LANES,), jnp.float32) for _ in range(D)))

                inv = 1.0 / jnp.maximum(len_v[...], 1.0)
                for d in range(D):
                    pt_v[pl.ds(d * GB, GB)] = acc[d] * inv
                gidx = wid * NG + g
                pltpu.sync_copy(pt_v, outm_hbm.at[f, gidx])
                return carry

            lax.fori_loop(0, NG, group_body, 0)

    return k


def kernel(x_single, x_multi, x_multi_vals, x_multi_lens,
           single_tables, multi_tables):
    NSF, V, D = single_tables.shape
    NMF = multi_tables.shape[0]
    B, _, L = x_multi.shape
    GB = 16

    ts = single_tables.reshape(NSF * V, D)
    tm = multi_tables.reshape(NMF * V, D)
    idx_s = (x_single.astype(jnp.int32)
             + jnp.arange(NSF, dtype=jnp.int32) * V).T.reshape(NSF * B)
    offs_m = (jnp.arange(NMF, dtype=jnp.int32) * V)[None, :, None]
    idx_m = jnp.transpose(x_multi.astype(jnp.int32) + offs_m,
                          (1, 0, 2)).reshape(NMF * B * L)
    vals_r = jnp.transpose(x_multi_vals, (1, 0, 2)).reshape(NMF * B * L)
    len_t = x_multi_lens.astype(jnp.float32).T.reshape(NMF * B)

    k = _make_kernel(B, NSF, NMF, L, V, D)
    outs, outm = k(ts, tm, idx_s, idx_m, vals_r, len_t)
    s = outs.transpose(1, 0, 2).reshape(B, NSF * D)
    m = outm.reshape(NMF, B // GB, D, GB).transpose(1, 3, 0, 2)
    return jnp.concatenate([s, m.reshape(B, NMF * D)], axis=1)


# speedup vs baseline: 3.4764x; 1.0187x over previous
import functools

import jax
import jax.numpy as jnp
from jax import lax
from jax.experimental import pallas as pl
from jax.experimental.pallas import tpu as pltpu
from jax.experimental.pallas import tpu_sc as plsc

NC = 2
NS = 16
LANES = 16
NW = NC * NS

IDXW = 80


def _make_kernel(B, NSF, NMF, L, V, D):
    RB = B // NW
    GB = 8
    NG = RB // GB
    NF = NSF + NMF
    NSP = (NSF + 7) // 8 * 8
    FL = NMF * L
    GFL = GB * FL
    NT = GB * NMF
    NBLK = NT // LANES
    KPG = GFL // IDXW
    KPB = KPG // NBLK

    mesh = plsc.VectorSubcoreMesh(core_axis_name="c", subcore_axis_name="s")

    @functools.partial(
        pl.kernel,
        out_type=jax.ShapeDtypeStruct((B * NF, D), jnp.float32),
        mesh=mesh,
        compiler_params=pltpu.CompilerParams(
            needs_layout_passes=False, use_tc_tiling_on_sc=False),
        scratch_types=[
            pltpu.VMEM((GB * NSP,), jnp.int32),
            pltpu.VMEM((GFL,), jnp.int32),
            pltpu.VMEM((GFL,), jnp.float32),
            pltpu.VMEM((GB * NMF,), jnp.int32),
            pltpu.VMEM((GFL, D), jnp.float32),
            pltpu.VMEM((GB * NF, D), jnp.float32),
            pltpu.SemaphoreType.DMA,
            pltpu.SemaphoreType.DMA,
            pltpu.SemaphoreType.DMA,
            pltpu.SemaphoreType.DMA,
        ],
    )
    def k(ts_hbm, tm_hbm, xs_hbm, xm_hbm, vals_hbm, len_hbm, out_hbm,
          sidxp_v, midx_v, vals_v, len_v, stage_v, rowbuf_v,
          sem0, sem1, sem2, sem_s):
        msems = [sem0, sem1, sem2]
        wid = lax.axis_index("s") * NC + lax.axis_index("c")
        base = pl.multiple_of(wid * RB, RB)

        iota = lax.iota(jnp.int32, LANES)

        def group_body(g, carry):
            gbase = pl.multiple_of(base + g * GB, GB)

            pltpu.sync_copy(xm_hbm.at[pl.ds(gbase * FL, GFL)], midx_v)
            pltpu.sync_copy(vals_hbm.at[pl.ds(gbase * FL, GFL)], vals_v)
            pltpu.sync_copy(len_hbm.at[pl.ds(gbase * NMF, GB * NMF)], len_v)
            pltpu.sync_copy(xs_hbm.at[pl.ds(gbase * NSP, GB * NSP)], sidxp_v)

            mcopies = [
                pltpu.async_copy(
                    tm_hbm.at[midx_v.at[pl.ds(kk * IDXW, IDXW)]],
                    stage_v.at[pl.ds(kk * IDXW, IDXW)], msems[kk // KPB])
                for kk in range(KPG)
            ]

            scopies = [
                pltpu.async_copy(
                    ts_hbm.at[sidxp_v.at[pl.ds(b * NSP, NSP)]],
                    rowbuf_v.at[pl.ds(b * NF, NSP)], sem_s)
                for b in range(GB)
            ]

            for j in range(NBLK):
                for c in mcopies[j * KPB:(j + 1) * KPB]:
                    c.wait()
                tvec = j * LANES + iota
                bvec = tvec // NMF
                fvec = tvec % NMF
                pad = fvec * V
                tb = tvec * L

                def l_body(l, acc, pad=pad, tb=tb):
                    jv = tb + l
                    iv = plsc.load_gather(midx_v, [jv])
                    wv = plsc.load_gather(vals_v, [jv])
                    wv = jnp.where(iv != pad, wv, 0.0)
                    out = []
                    for d in range(D):
                        dcol = jnp.full((LANES,), d, jnp.int32)
                        gv = plsc.load_gather(stage_v, [jv, dcol])
                        out.append(acc[d] + wv * gv)
                    return tuple(out)

                acc = lax.fori_loop(
                    0, L, l_body,
                    tuple(jnp.zeros((LANES,), jnp.float32) for _ in range(D)))

                lv = plsc.load_gather(len_v, [tvec]).astype(jnp.float32)
                inv = 1.0 / jnp.maximum(lv, 1.0)
                if j == 0:
                    for c in scopies:
                        c.wait()
                row = bvec * NF + NSF + fvec
                for d in range(D):
                    dcol = jnp.full((LANES,), d, jnp.int32)
                    plsc.store_scatter(rowbuf_v, [row, dcol], acc[d] * inv)

            pltpu.sync_copy(
                rowbuf_v, out_hbm.at[pl.ds(gbase * NF, GB * NF)])
            return carry

        lax.fori_loop(0, NG, group_body, 0)

    return k


def kernel(x_single, x_multi, x_multi_vals, x_multi_lens,
           single_tables, multi_tables):
    NSF, V, D = single_tables.shape
    NMF = multi_tables.shape[0]
    B, _, L = x_multi.shape

    NSP = (NSF + 7) // 8 * 8
    idx_s = x_single.astype(jnp.int32) + jnp.arange(NSF, dtype=jnp.int32) * V
    idx_s = jnp.concatenate(
        [idx_s, jnp.zeros((B, NSP - NSF), jnp.int32)], axis=1)
    idx_m = (x_multi.astype(jnp.int32)
             + (jnp.arange(NMF, dtype=jnp.int32) * V)[None, :, None])

    k = _make_kernel(B, NSF, NMF, L, V, D)
    out = k(single_tables.reshape(NSF * V, D),
            multi_tables.reshape(NMF * V, D),
            idx_s.reshape(B * NSP),
            idx_m.reshape(B * NMF * L),
            x_multi_vals.reshape(B * NMF * L),
            x_multi_lens.astype(jnp.int32).reshape(B * NMF))
    return out.reshape(B, (NSF + NMF) * D)


# speedup vs baseline: 3.4785x; 1.0006x over previous
import functools

import jax
import jax.numpy as jnp
from jax import lax
from jax.experimental import pallas as pl
from jax.experimental.pallas import tpu as pltpu
from jax.experimental.pallas import tpu_sc as plsc

NC = 2
NS = 16
LANES = 16
NW = NC * NS

IDXW = 800


def _make_kernel(B, NSF, NMF, L, V, D):
    RB = B // NW
    GB = 8
    NG = RB // GB
    NF = NSF + NMF
    NSP = (NSF + 7) // 8 * 8
    FL = NMF * L
    GFL = GB * FL
    NT = GB * NMF
    NBLK = NT // LANES
    KPG = GFL // IDXW
    KPB = KPG // NBLK

    mesh = plsc.VectorSubcoreMesh(core_axis_name="c", subcore_axis_name="s")

    @functools.partial(
        pl.kernel,
        out_type=jax.ShapeDtypeStruct((B * NF, D), jnp.float32),
        mesh=mesh,
        compiler_params=pltpu.CompilerParams(
            needs_layout_passes=False, use_tc_tiling_on_sc=False),
        scratch_types=[
            pltpu.VMEM((GB * NSP,), jnp.int32),
            pltpu.VMEM((GFL,), jnp.int32),
            pltpu.VMEM((GFL,), jnp.float32),
            pltpu.VMEM((GB * NMF,), jnp.int32),
            pltpu.VMEM((GFL, D), jnp.float32),
            pltpu.VMEM((GB * NF, D), jnp.float32),
            pltpu.SemaphoreType.DMA,
            pltpu.SemaphoreType.DMA,
            pltpu.SemaphoreType.DMA,
            pltpu.SemaphoreType.DMA,
        ],
    )
    def k(ts_hbm, tm_hbm, xs_hbm, xm_hbm, vals_hbm, len_hbm, out_hbm,
          sidxp_v, midx_v, vals_v, len_v, stage_v, rowbuf_v,
          sem0, sem1, sem2, sem_s):
        msems = [sem0, sem1, sem2]
        wid = lax.axis_index("s") * NC + lax.axis_index("c")
        base = pl.multiple_of(wid * RB, RB)

        iota = lax.iota(jnp.int32, LANES)

        def group_body(g, carry):
            gbase = pl.multiple_of(base + g * GB, GB)

            pltpu.sync_copy(xm_hbm.at[pl.ds(gbase * FL, GFL)], midx_v)
            pltpu.sync_copy(vals_hbm.at[pl.ds(gbase * FL, GFL)], vals_v)
            pltpu.sync_copy(len_hbm.at[pl.ds(gbase * NMF, GB * NMF)], len_v)
            pltpu.sync_copy(xs_hbm.at[pl.ds(gbase * NSP, GB * NSP)], sidxp_v)

            mcopies = [
                pltpu.async_copy(
                    tm_hbm.at[midx_v.at[pl.ds(kk * IDXW, IDXW)]],
                    stage_v.at[pl.ds(kk * IDXW, IDXW)], msems[kk // KPB])
                for kk in range(KPG)
            ]

            scopies = [
                pltpu.async_copy(
                    ts_hbm.at[sidxp_v.at[pl.ds(b * NSP, NSP)]],
                    rowbuf_v.at[pl.ds(b * NF, NSP)], sem_s)
                for b in range(GB)
            ]

            for j in range(NBLK):
                for c in mcopies[j * KPB:(j + 1) * KPB]:
                    c.wait()
                tvec = j * LANES + iota
                bvec = tvec // NMF
                fvec = tvec % NMF
                pad = fvec * V
                tb = tvec * L

                def l_body(l, acc, pad=pad, tb=tb):
                    jv = tb + l
                    iv = plsc.load_gather(midx_v, [jv])
                    wv = plsc.load_gather(vals_v, [jv])
                    wv = jnp.where(iv != pad, wv, 0.0)
                    out = []
                    for d in range(D):
                        dcol = jnp.full((LANES,), d, jnp.int32)
                        gv = plsc.load_gather(stage_v, [jv, dcol])
                        out.append(acc[d] + wv * gv)
                    return tuple(out)

                acc = lax.fori_loop(
                    0, L, l_body,
                    tuple(jnp.zeros((LANES,), jnp.float32) for _ in range(D)))

                lv = plsc.load_gather(len_v, [tvec]).astype(jnp.float32)
                inv = 1.0 / jnp.maximum(lv, 1.0)
                if j == 0:
                    for c in scopies:
                        c.wait()
                row = bvec * NF + NSF + fvec
                for d in range(D):
                    dcol = jnp.full((LANES,), d, jnp.int32)
                    plsc.store_scatter(rowbuf_v, [row, dcol], acc[d] * inv)

            pltpu.sync_copy(
                rowbuf_v, out_hbm.at[pl.ds(gbase * NF, GB * NF)])
            return carry

        lax.fori_loop(0, NG, group_body, 0)

    return k


def kernel(x_single, x_multi, x_multi_vals, x_multi_lens,
           single_tables, multi_tables):
    NSF, V, D = single_tables.shape
    NMF = multi_tables.shape[0]
    B, _, L = x_multi.shape

    NSP = (NSF + 7) // 8 * 8
    idx_s = x_single.astype(jnp.int32) + jnp.arange(NSF, dtype=jnp.int32) * V
    idx_s = jnp.concatenate(
        [idx_s, jnp.zeros((B, NSP - NSF), jnp.int32)], axis=1)
    idx_m = (x_multi.astype(jnp.int32)
             + (jnp.arange(NMF, dtype=jnp.int32) * V)[None, :, None])

    k = _make_kernel(B, NSF, NMF, L, V, D)
    out = k(single_tables.reshape(NSF * V, D),
            multi_tables.reshape(NMF * V, D),
            idx_s.reshape(B * NSP),
            idx_m.reshape(B * NMF * L),
            x_multi_vals.reshape(B * NMF * L),
            x_multi_lens.astype(jnp.int32).reshape(B * NMF))
    return out.reshape(B, (NSF + NMF) * D)


# speedup vs baseline: 5.4796x; 1.5753x over previous
import functools

import jax
import jax.numpy as jnp
from jax import lax
from jax.experimental import pallas as pl
from jax.experimental.pallas import tpu as pltpu
from jax.experimental.pallas import tpu_sc as plsc

NC = 2
NS = 16
LANES = 16
NW = NC * NS

_PARAMS = pltpu.CompilerParams(
    needs_layout_passes=False, use_tc_tiling_on_sc=False)
_MESH = dict(core_axis_name="c", subcore_axis_name="s")


def _make_multi_kernel(B, NMF, L, V, D):
    RB = B // NW
    GB = 8
    NG = RB // GB
    FL = NMF * L
    GFL = GB * FL
    NT = GB * NMF
    NBLK = NT // LANES
    SEC = GFL // NBLK

    @functools.partial(
        pl.kernel,
        out_type=jax.ShapeDtypeStruct((B * NMF, D), jnp.float32),
        mesh=plsc.VectorSubcoreMesh(**_MESH),
        compiler_params=_PARAMS,
        scratch_types=[
            pltpu.VMEM((GFL,), jnp.int32),
            pltpu.VMEM((GFL,), jnp.float32),
            pltpu.VMEM((GB * NMF,), jnp.int32),
            pltpu.VMEM((GFL, D), jnp.float32),
            pltpu.VMEM((NT, D), jnp.float32),
            pltpu.SemaphoreType.DMA,
            pltpu.SemaphoreType.DMA,
            pltpu.SemaphoreType.DMA,
        ],
    )
    def k(tm_hbm, xm_hbm, vals_hbm, len_hbm, out_hbm,
          midx_v, vals_v, len_v, stage_v, pool_v, sem0, sem1, sem2):
        msems = [sem0, sem1, sem2]
        wid = lax.axis_index("s") * NC + lax.axis_index("c")
        base = pl.multiple_of(wid * RB, RB)
        iota = lax.iota(jnp.int32, LANES)

        def group_body(g, carry):
            gbase = pl.multiple_of(base + g * GB, GB)
            pltpu.sync_copy(xm_hbm.at[pl.ds(gbase * FL, GFL)], midx_v)
            pltpu.sync_copy(vals_hbm.at[pl.ds(gbase * FL, GFL)], vals_v)
            pltpu.sync_copy(len_hbm.at[pl.ds(gbase * NMF, GB * NMF)], len_v)

            mcopies = [
                pltpu.async_copy(
                    tm_hbm.at[midx_v.at[pl.ds(kk * SEC, SEC)]],
                    stage_v.at[pl.ds(kk * SEC, SEC)], msems[kk])
                for kk in range(NBLK)
            ]

            for j in range(NBLK):
                mcopies[j].wait()
                tvec = j * LANES + iota
                pad = tvec % NMF * V
                tb = tvec * L

                def l_body(l, acc, pad=pad, tb=tb):
                    jv = tb + l
                    iv = plsc.load_gather(midx_v, [jv])
                    wv = plsc.load_gather(vals_v, [jv])
                    wv = jnp.where(iv != pad, wv, 0.0)
                    out = []
                    for d in range(D):
                        dcol = jnp.full((LANES,), d, jnp.int32)
                        gv = plsc.load_gather(stage_v, [jv, dcol])
                        out.append(acc[d] + wv * gv)
                    return tuple(out)

                acc = lax.fori_loop(
                    0, L, l_body,
                    tuple(jnp.zeros((LANES,), jnp.float32) for _ in range(D)))

                lv = plsc.load_gather(len_v, [tvec]).astype(jnp.float32)
                inv = 1.0 / jnp.maximum(lv, 1.0)
                for d in range(D):
                    dcol = jnp.full((LANES,), d, jnp.int32)
                    plsc.store_scatter(pool_v, [tvec, dcol], acc[d] * inv)

            pltpu.sync_copy(pool_v, out_hbm.at[pl.ds(gbase * NMF, NT)])
            return carry

        lax.fori_loop(0, NG, group_body, 0)

    return k


def _make_single_kernel(B, NSF, V, D):
    RB = B // NW
    PR = 2
    NSTR = RB // PR

    @functools.partial(
        pl.kernel,
        out_type=jax.ShapeDtypeStruct((B * NSF, D), jnp.float32),
        mesh=plsc.VectorSubcoreMesh(**_MESH),
        compiler_params=_PARAMS,
        scratch_types=[
            pltpu.VMEM((RB * NSF,), jnp.int32),
            pltpu.VMEM((RB * NSF, D), jnp.float32),
            pltpu.SemaphoreType.DMA,
        ],
    )
    def k(ts_hbm, xs_hbm, out_hbm, sidx_v, rows_v, sem):
        wid = lax.axis_index("s") * NC + lax.axis_index("c")
        base = pl.multiple_of(wid * RB, RB)
        pltpu.sync_copy(xs_hbm.at[pl.ds(base * NSF, RB * NSF)], sidx_v)
        copies = [
            pltpu.async_copy(
                ts_hbm.at[sidx_v.at[pl.ds(p * PR * NSF, PR * NSF)]],
                rows_v.at[pl.ds(p * PR * NSF, PR * NSF)], sem)
            for p in range(NSTR)
        ]
        for c in copies:
            c.wait()
        pltpu.sync_copy(rows_v, out_hbm.at[pl.ds(base * NSF, RB * NSF)])

    return k


def kernel(x_single, x_multi, x_multi_vals, x_multi_lens,
           single_tables, multi_tables):
    NSF, V, D = single_tables.shape
    NMF = multi_tables.shape[0]
    B, _, L = x_multi.shape

    idx_s = x_single.astype(jnp.int32) + jnp.arange(NSF, dtype=jnp.int32) * V
    idx_m = (x_multi.astype(jnp.int32)
             + (jnp.arange(NMF, dtype=jnp.int32) * V)[None, :, None])

    km = _make_multi_kernel(B, NMF, L, V, D)
    out_m = km(multi_tables.reshape(NMF * V, D),
               idx_m.reshape(B * NMF * L),
               x_multi_vals.reshape(B * NMF * L),
               x_multi_lens.astype(jnp.int32).reshape(B * NMF))
    ks = _make_single_kernel(B, NSF, V, D)
    out_s = ks(single_tables.reshape(NSF * V, D), idx_s.reshape(B * NSF))
    return jnp.concatenate(
        [out_s.reshape(B, NSF * D), out_m.reshape(B, NMF * D)], axis=1)


# speedup vs baseline: 5.4851x; 1.0010x over previous
import functools

import jax
import jax.numpy as jnp
from jax import lax
from jax.experimental import pallas as pl
from jax.experimental.pallas import tpu as pltpu
from jax.experimental.pallas import tpu_sc as plsc

NC = 2
NS = 16
LANES = 16
NW = NC * NS

_PARAMS = pltpu.CompilerParams(
    needs_layout_passes=False, use_tc_tiling_on_sc=False)
_MESH = dict(core_axis_name="c", subcore_axis_name="s")


def _make_multi_kernel(B, NMF, L, V, D):
    RB = B // NW
    GB = 8
    NG = RB // GB
    FL = NMF * L
    GFL = GB * FL
    NT = GB * NMF
    NBLK = NT // LANES
    SEC = GFL // NBLK

    @functools.partial(
        pl.kernel,
        out_type=jax.ShapeDtypeStruct((B * NMF, D), jnp.float32),
        mesh=plsc.VectorSubcoreMesh(**_MESH),
        compiler_params=_PARAMS,
        scratch_types=[
            pltpu.VMEM((GFL,), jnp.int32),
            pltpu.VMEM((GFL,), jnp.float32),
            pltpu.VMEM((GB * NMF,), jnp.int32),
            pltpu.VMEM((GFL, D), jnp.float32),
            pltpu.VMEM((NT, D), jnp.float32),
            pltpu.SemaphoreType.DMA,
            pltpu.SemaphoreType.DMA,
            pltpu.SemaphoreType.DMA,
        ],
    )
    def k(tm_hbm, xm_hbm, vals_hbm, len_hbm, out_hbm,
          midx_v, vals_v, len_v, stage_v, pool_v, sem0, sem1, sem2):
        msems = [sem0, sem1, sem2]
        wid = lax.axis_index("s") * NC + lax.axis_index("c")
        base = pl.multiple_of(wid * RB, RB)
        iota = lax.iota(jnp.int32, LANES)

        def group_body(g, carry):
            gbase = pl.multiple_of(base + g * GB, GB)
            pltpu.sync_copy(xm_hbm.at[pl.ds(gbase * FL, GFL)], midx_v)
            pltpu.sync_copy(vals_hbm.at[pl.ds(gbase * FL, GFL)], vals_v)
            pltpu.sync_copy(len_hbm.at[pl.ds(gbase * NMF, GB * NMF)], len_v)

            mcopies = [
                pltpu.async_copy(
                    tm_hbm.at[midx_v.at[pl.ds(kk * SEC, SEC)]],
                    stage_v.at[pl.ds(kk * SEC, SEC)], msems[kk])
                for kk in range(NBLK)
            ]

            for j in range(NBLK):
                mcopies[j].wait()
                tvec = j * LANES + iota
                pad = tvec % NMF * V
                tb = tvec * L

                def l_body(l, acc, pad=pad, tb=tb):
                    jv = tb + l
                    iv = plsc.load_gather(midx_v, [jv])
                    wv = plsc.load_gather(vals_v, [jv])
                    wv = jnp.where(iv != pad, wv, 0.0)
                    dvec = jnp.zeros((LANES,), jnp.int32)
                    out = []
                    for d in range(D):
                        gv = plsc.load_gather(stage_v, [jv, dvec])
                        dvec = dvec + 1
                        out.append(acc[d] + wv * gv)
                    return tuple(out)

                acc = lax.fori_loop(
                    0, L, l_body,
                    tuple(jnp.zeros((LANES,), jnp.float32) for _ in range(D)))

                lv = plsc.load_gather(len_v, [tvec]).astype(jnp.float32)
                inv = 1.0 / jnp.maximum(lv, 1.0)
                dvec = jnp.zeros((LANES,), jnp.int32)
                for d in range(D):
                    plsc.store_scatter(pool_v, [tvec, dvec], acc[d] * inv)
                    dvec = dvec + 1

            pltpu.sync_copy(pool_v, out_hbm.at[pl.ds(gbase * NMF, NT)])
            return carry

        lax.fori_loop(0, NG, group_body, 0)

    return k


def _make_single_kernel(B, NSF, V, D):
    RB = B // NW
    PR = 2
    NSTR = RB // PR

    @functools.partial(
        pl.kernel,
        out_type=jax.ShapeDtypeStruct((B * NSF, D), jnp.float32),
        mesh=plsc.VectorSubcoreMesh(**_MESH),
        compiler_params=_PARAMS,
        scratch_types=[
            pltpu.VMEM((RB * NSF,), jnp.int32),
            pltpu.VMEM((RB * NSF, D), jnp.float32),
            pltpu.SemaphoreType.DMA,
        ],
    )
    def k(ts_hbm, xs_hbm, out_hbm, sidx_v, rows_v, sem):
        wid = lax.axis_index("s") * NC + lax.axis_index("c")
        base = pl.multiple_of(wid * RB, RB)
        pltpu.sync_copy(xs_hbm.at[pl.ds(base * NSF, RB * NSF)], sidx_v)
        copies = [
            pltpu.async_copy(
                ts_hbm.at[sidx_v.at[pl.ds(p * PR * NSF, PR * NSF)]],
                rows_v.at[pl.ds(p * PR * NSF, PR * NSF)], sem)
            for p in range(NSTR)
        ]
        for c in copies:
            c.wait()
        pltpu.sync_copy(rows_v, out_hbm.at[pl.ds(base * NSF, RB * NSF)])

    return k


def kernel(x_single, x_multi, x_multi_vals, x_multi_lens,
           single_tables, multi_tables):
    NSF, V, D = single_tables.shape
    NMF = multi_tables.shape[0]
    B, _, L = x_multi.shape

    idx_s = x_single.astype(jnp.int32) + jnp.arange(NSF, dtype=jnp.int32) * V
    idx_m = (x_multi.astype(jnp.int32)
             + (jnp.arange(NMF, dtype=jnp.int32) * V)[None, :, None])

    km = _make_multi_kernel(B, NMF, L, V, D)
    out_m = km(multi_tables.reshape(NMF * V, D),
               idx_m.reshape(B * NMF * L),
               x_multi_vals.reshape(B * NMF * L),
               x_multi_lens.astype(jnp.int32).reshape(B * NMF))
    ks = _make_single_kernel(B, NSF, V, D)
    out_s = ks(single_tables.reshape(NSF * V, D), idx_s.reshape(B * NSF))
    return jnp.concatenate(
        [out_s.reshape(B, NSF * D), out_m.reshape(B, NMF * D)], axis=1)


# speedup vs baseline: 5.5709x; 1.0157x over previous
import functools

import jax
import jax.numpy as jnp
from jax import lax
from jax.experimental import pallas as pl
from jax.experimental.pallas import tpu as pltpu
from jax.experimental.pallas import tpu_sc as plsc

NC = 2
NS = 16
LANES = 16
NW = NC * NS

_PARAMS = pltpu.CompilerParams(
    needs_layout_passes=False, use_tc_tiling_on_sc=False)
_MESH = dict(core_axis_name="c", subcore_axis_name="s")


def _make_multi_kernel(B, NMF, L, V, D):
    RB = B // NW
    NT = RB * NMF
    NB = NT // LANES
    SEC = LANES * L

    @functools.partial(
        pl.kernel,
        out_type=jax.ShapeDtypeStruct((B * NMF, D), jnp.float32),
        mesh=plsc.VectorSubcoreMesh(**_MESH),
        compiler_params=_PARAMS,
        scratch_types=[
            [pltpu.VMEM((SEC,), jnp.int32)] * 2,
            [pltpu.VMEM((SEC,), jnp.float32)] * 2,
            pltpu.VMEM((NT,), jnp.int32),
            [pltpu.VMEM((SEC, D), jnp.float32)] * 2,
            [pltpu.VMEM((LANES, D), jnp.float32)] * 2,
            [pltpu.SemaphoreType.DMA] * 6,
        ],
    )
    def k(tm_hbm, xm_hbm, vals_hbm, len_hbm, out_hbm,
          midx_r, vals_r, len_v, stage_r, pool_r, sems):
        sem_in = sems[0:2]
        sem_st = sems[2:4]
        sem_out = sems[4:6]
        wid = lax.axis_index("s") * NC + lax.axis_index("c")
        base = pl.multiple_of(wid * RB, RB)
        fbase = base * NMF * L
        iota = lax.iota(jnp.int32, LANES)
        tb = iota * L

        def issue_in(t, p):
            tc = jnp.minimum(t, NB - 2 + p)
            off = pl.multiple_of(fbase + tc * SEC, SEC)
            pltpu.async_copy(xm_hbm.at[pl.ds(off, SEC)], midx_r[p], sem_in[p])
            pltpu.async_copy(vals_hbm.at[pl.ds(off, SEC)], vals_r[p],
                             sem_in[p])

        def wait_in(p):
            pltpu.make_async_copy(xm_hbm.at[pl.ds(0, SEC)], midx_r[p],
                                  sem_in[p]).wait()
            pltpu.make_async_copy(vals_hbm.at[pl.ds(0, SEC)], vals_r[p],
                                  sem_in[p]).wait()

        def issue_stream(p):
            pltpu.async_copy(tm_hbm.at[midx_r[p]], stage_r[p], sem_st[p])

        def wait_stream(p):
            pltpu.make_async_copy(tm_hbm.at[pl.ds(0, SEC)], stage_r[p],
                                  sem_st[p]).wait()

        def issue_out(t, p):
            pltpu.async_copy(
                pool_r[p],
                out_hbm.at[pl.ds(pl.multiple_of(
                    base * NMF + t * LANES, LANES), LANES)],
                sem_out[p])

        def wait_out(p):
            pltpu.make_async_copy(pool_r[p], out_hbm.at[pl.ds(0, LANES)],
                                  sem_out[p]).wait()

        def compute(t, p):
            tglob = t * LANES + iota
            pad = tglob % NMF * V

            def l_body(l, acc, pad=pad):
                jv = tb + l
                iv = plsc.load_gather(midx_r[p], [jv])
                wv = plsc.load_gather(vals_r[p], [jv])
                wv = jnp.where(iv != pad, wv, 0.0)
                dvec = jnp.zeros((LANES,), jnp.int32)
                out = []
                for d in range(D):
                    gv = plsc.load_gather(stage_r[p], [jv, dvec])
                    dvec = dvec + 1
                    out.append(acc[d] + wv * gv)
                return tuple(out)

            acc = lax.fori_loop(
                0, L, l_body,
                tuple(jnp.zeros((LANES,), jnp.float32) for _ in range(D)))

            lv = plsc.load_gather(len_v, [tglob]).astype(jnp.float32)
            inv = 1.0 / jnp.maximum(lv, 1.0)
            wait_out(p)
            dvec = jnp.zeros((LANES,), jnp.int32)
            for d in range(D):
                plsc.store_scatter(pool_r[p], [iota, dvec], acc[d] * inv)
                dvec = dvec + 1
            issue_out(t, p)

        pltpu.sync_copy(len_hbm.at[pl.ds(base * NMF, NT)], len_v)
        issue_in(0, 0)
        issue_in(1, 1)
        issue_out(0, 0)
        issue_out(1, 1)

        def pipe_body(i, carry):
            t0 = pl.multiple_of(i * 2, 2)
            t1 = t0 + 1
            wait_in(0)
            issue_stream(0)
            wait_in(1)
            issue_stream(1)
            wait_stream(0)
            compute(t0, 0)
            issue_in(t0 + 2, 0)
            wait_stream(1)
            compute(t1, 1)
            issue_in(t1 + 2, 1)
            return carry

        lax.fori_loop(0, NB // 2, pipe_body, 0)

        wait_in(0)
        wait_in(1)
        wait_out(0)
        wait_out(1)

    return k


def _make_single_kernel(B, NSF, V, D):
    RB = B // NW
    PR = 2
    NSTR = RB // PR

    @functools.partial(
        pl.kernel,
        out_type=jax.ShapeDtypeStruct((B * NSF, D), jnp.float32),
        mesh=plsc.VectorSubcoreMesh(**_MESH),
        compiler_params=_PARAMS,
        scratch_types=[
            pltpu.VMEM((RB * NSF,), jnp.int32),
            pltpu.VMEM((RB * NSF, D), jnp.float32),
            pltpu.SemaphoreType.DMA,
        ],
    )
    def k(ts_hbm, xs_hbm, out_hbm, sidx_v, rows_v, sem):
        wid = lax.axis_index("s") * NC + lax.axis_index("c")
        base = pl.multiple_of(wid * RB, RB)
        pltpu.sync_copy(xs_hbm.at[pl.ds(base * NSF, RB * NSF)], sidx_v)
        copies = [
            pltpu.async_copy(
                ts_hbm.at[sidx_v.at[pl.ds(p * PR * NSF, PR * NSF)]],
                rows_v.at[pl.ds(p * PR * NSF, PR * NSF)], sem)
            for p in range(NSTR)
        ]
        for c in copies:
            c.wait()
        pltpu.sync_copy(rows_v, out_hbm.at[pl.ds(base * NSF, RB * NSF)])

    return k


def kernel(x_single, x_multi, x_multi_vals, x_multi_lens,
           single_tables, multi_tables):
    NSF, V, D = single_tables.shape
    NMF = multi_tables.shape[0]
    B, _, L = x_multi.shape

    idx_s = x_single.astype(jnp.int32) + jnp.arange(NSF, dtype=jnp.int32) * V
    idx_m = (x_multi.astype(jnp.int32)
             + (jnp.arange(NMF, dtype=jnp.int32) * V)[None, :, None])

    km = _make_multi_kernel(B, NMF, L, V, D)
    out_m = km(multi_tables.reshape(NMF * V, D),
               idx_m.reshape(B * NMF * L),
               x_multi_vals.reshape(B * NMF * L),
               x_multi_lens.astype(jnp.int32).reshape(B * NMF))
    ks = _make_single_kernel(B, NSF, V, D)
    out_s = ks(single_tables.reshape(NSF * V, D), idx_s.reshape(B * NSF))
    return jnp.concatenate(
        [out_s.reshape(B, NSF * D), out_m.reshape(B, NMF * D)], axis=1)


# speedup vs baseline: 5.5758x; 1.0009x over previous
import functools

import jax
import jax.numpy as jnp
from jax import lax
from jax.experimental import pallas as pl
from jax.experimental.pallas import tpu as pltpu
from jax.experimental.pallas import tpu_sc as plsc

NC = 2
NS = 16
LANES = 16
NW = NC * NS

_PARAMS = pltpu.CompilerParams(
    needs_layout_passes=False, use_tc_tiling_on_sc=False)
_MESH = dict(core_axis_name="c", subcore_axis_name="s")


def _make_multi_kernel(B, NMF, L, V, D):
    RB = B // NW
    NT = RB * NMF
    NB = NT // LANES
    SEC = LANES * L

    @functools.partial(
        pl.kernel,
        out_type=jax.ShapeDtypeStruct((B * NMF, D), jnp.float32),
        mesh=plsc.VectorSubcoreMesh(**_MESH),
        compiler_params=_PARAMS,
        scratch_types=[
            [pltpu.VMEM((SEC,), jnp.int32)] * 2,
            [pltpu.VMEM((SEC,), jnp.float32)] * 2,
            pltpu.VMEM((NT,), jnp.int32),
            [pltpu.VMEM((SEC, D), jnp.float32)] * 2,
            [pltpu.VMEM((LANES, D), jnp.float32)] * 2,
            [pltpu.SemaphoreType.DMA] * 6,
        ],
    )
    def k(tm_hbm, xm_hbm, vals_hbm, len_hbm, out_hbm,
          midx_r, vals_r, len_v, stage_r, pool_r, sems):
        sem_in = sems[0:2]
        sem_st = sems[2:4]
        sem_out = sems[4:6]
        wid = lax.axis_index("s") * NC + lax.axis_index("c")
        base = pl.multiple_of(wid * RB, RB)
        fbase = base * NMF * L
        iota = lax.iota(jnp.int32, LANES)
        tb = iota * L

        def issue_in(t, p):
            tc = jnp.minimum(t, NB - 2 + p)
            off = pl.multiple_of(fbase + tc * SEC, SEC)
            pltpu.async_copy(xm_hbm.at[pl.ds(off, SEC)], midx_r[p], sem_in[p])
            pltpu.async_copy(vals_hbm.at[pl.ds(off, SEC)], vals_r[p],
                             sem_in[p])

        def wait_in(p):
            pltpu.make_async_copy(xm_hbm.at[pl.ds(0, SEC)], midx_r[p],
                                  sem_in[p]).wait()
            pltpu.make_async_copy(vals_hbm.at[pl.ds(0, SEC)], vals_r[p],
                                  sem_in[p]).wait()

        def issue_stream(p):
            pltpu.async_copy(tm_hbm.at[midx_r[p]], stage_r[p], sem_st[p])

        def wait_stream(p):
            pltpu.make_async_copy(tm_hbm.at[pl.ds(0, SEC)], stage_r[p],
                                  sem_st[p]).wait()

        def issue_out(t, p):
            pltpu.async_copy(
                pool_r[p],
                out_hbm.at[pl.ds(pl.multiple_of(
                    base * NMF + t * LANES, LANES), LANES)],
                sem_out[p])

        def wait_out(p):
            pltpu.make_async_copy(pool_r[p], out_hbm.at[pl.ds(0, LANES)],
                                  sem_out[p]).wait()

        def compute(t, p):
            tglob = t * LANES + iota
            pad = tglob % NMF * V

            def l_body(l, acc, pad=pad):
                jv = tb + l
                iv = plsc.load_gather(midx_r[p], [jv])
                wv = plsc.load_gather(vals_r[p], [jv])
                wv = jnp.where(iv != pad, wv, 0.0)
                dvec = jnp.zeros((LANES,), jnp.int32)
                out = []
                for d in range(D):
                    gv = plsc.load_gather(stage_r[p], [jv, dvec])
                    dvec = dvec + 1
                    out.append(acc[d] + wv * gv)
                return tuple(out)

            acc = lax.fori_loop(
                0, L, l_body,
                tuple(jnp.zeros((LANES,), jnp.float32) for _ in range(D)))

            lv = plsc.load_gather(len_v, [tglob]).astype(jnp.float32)
            inv = 1.0 / jnp.maximum(lv, 1.0)
            wait_out(p)
            dvec = jnp.zeros((LANES,), jnp.int32)
            for d in range(D):
                plsc.store_scatter(pool_r[p], [iota, dvec], acc[d] * inv)
                dvec = dvec + 1
            issue_out(t, p)

        pltpu.sync_copy(len_hbm.at[pl.ds(base * NMF, NT)], len_v)
        issue_in(0, 0)
        issue_in(1, 1)
        issue_out(0, 0)
        issue_out(1, 1)

        def pipe_body(i, carry):
            t0 = pl.multiple_of(i * 2, 2)
            t1 = t0 + 1
            wait_in(0)
            issue_stream(0)
            wait_in(1)
            issue_stream(1)
            wait_stream(0)
            compute(t0, 0)
            issue_in(t0 + 2, 0)
            wait_stream(1)
            compute(t1, 1)
            issue_in(t1 + 2, 1)
            return carry

        lax.fori_loop(0, NB // 2, pipe_body, 0)

        wait_in(0)
        wait_in(1)
        wait_out(0)
        wait_out(1)

    return k


def _make_single_kernel(B, NSF, V, D):
    RB = B // NW
    PR = 2
    NSTR = RB // PR

    @functools.partial(
        pl.kernel,
        out_type=jax.ShapeDtypeStruct((B * NSF, D), jnp.float32),
        mesh=plsc.VectorSubcoreMesh(**_MESH),
        compiler_params=_PARAMS,
        scratch_types=[
            pltpu.VMEM((RB * NSF,), jnp.int32),
            pltpu.VMEM((RB * NSF, D), jnp.float32),
            pltpu.SemaphoreType.DMA,
        ],
    )
    def k(ts_hbm, xs_hbm, out_hbm, sidx_v, rows_v, sem):
        wid = lax.axis_index("s") * NC + lax.axis_index("c")
        base = pl.multiple_of(wid * RB, RB)
        pltpu.sync_copy(xs_hbm.at[pl.ds(base * NSF, RB * NSF)], sidx_v)
        copies = [
            pltpu.async_copy(
                ts_hbm.at[sidx_v.at[pl.ds(p * PR * NSF, PR * NSF)]],
                rows_v.at[pl.ds(p * PR * NSF, PR * NSF)], sem)
            for p in range(NSTR)
        ]
        for c in copies:
            c.wait()
        pltpu.sync_copy(rows_v, out_hbm.at[pl.ds(base * NSF, RB * NSF)])

    return k


def kernel(x_single, x_multi, x_multi_vals, x_multi_lens,
           single_tables, multi_tables):
    NSF, V, D = single_tables.shape
    NMF = multi_tables.shape[0]
    B, _, L = x_multi.shape

    idx_s = x_single.astype(jnp.int32) + jnp.arange(NSF, dtype=jnp.int32) * V
    idx_m = (x_multi.astype(jnp.int32)
             + (jnp.arange(NMF, dtype=jnp.int32) * V)[None, :, None])

    def _relayout(t):
        tv = lax.optimization_barrier(jnp.swapaxes(t, 1, 2))
        return jnp.transpose(tv, (0, 2, 1))

    single_tables = _relayout(single_tables)
    multi_tables = _relayout(multi_tables)

    km = _make_multi_kernel(B, NMF, L, V, D)
    out_m = km(multi_tables.reshape(NMF * V, D),
               idx_m.reshape(B * NMF * L),
               x_multi_vals.reshape(B * NMF * L),
               x_multi_lens.astype(jnp.int32).reshape(B * NMF))
    ks = _make_single_kernel(B, NSF, V, D)
    out_s = ks(single_tables.reshape(NSF * V, D), idx_s.reshape(B * NSF))
    return jnp.concatenate(
        [out_s.reshape(B, NSF * D), out_m.reshape(B, NMF * D)], axis=1)
